# baseline (speedup 1.0000x reference)
import functools

import jax
import jax.numpy as jnp
from jax.experimental import pallas as pl
from jax.experimental.pallas import tpu as pltpu
from jax.experimental.pallas import tpu_sc as plsc

_WINDOW = 128


def kernel(x, emb):
    B, H = x.shape
    V, D = emb.shape
    n = B * H
    assert n % _WINDOW == 0
    idx = x.reshape(1, n).astype(jnp.int32)

    mesh = plsc.VectorSubcoreMesh(core_axis_name="core", subcore_axis_name="subcore")

    @functools.partial(
        pl.kernel,
        out_type=jax.ShapeDtypeStruct((n, D), emb.dtype),
        mesh=mesh,
        compiler_params=pltpu.CompilerParams(use_tc_tiling_on_sc=False),
    )
    def gather_kernel(emb_hbm, idx_hbm, out_hbm):
        def body(i_vmem, o_vmem):
            pltpu.sync_copy(emb_hbm.at[i_vmem.at[0]], o_vmem)

        pltpu.emit_pipeline(
            body,
            grid=(n // _WINDOW,),
            in_specs=[pl.BlockSpec((1, _WINDOW), index_map=lambda i: (0, i))],
            out_specs=[pl.BlockSpec((_WINDOW, D), index_map=lambda i: (i, 0))],
            core_axis_name=("core", "subcore"),
            dimension_semantics=(pltpu.PARALLEL,),
        )(idx_hbm, out_hbm)

    out = gather_kernel(emb, idx)
    return out.reshape(B, H, D)


# speedup vs baseline: 1.0750x; 1.0750x over previous
import functools

import jax
import jax.numpy as jnp
from jax import lax
from jax.experimental import pallas as pl
from jax.experimental.pallas import tpu as pltpu
from jax.experimental.pallas import tpu_sc as plsc

_W = 128
_NBUF = 4


def kernel(x, emb):
    B, H = x.shape
    V, D = emb.shape
    n = B * H
    info = plsc.get_sparse_core_info()
    nw = info.num_cores * info.num_subcores
    n_per_w = n // nw
    nsteps = n_per_w // _W
    assert n == nw * nsteps * _W and nsteps % _NBUF == 0

    idx3 = x.reshape(nw, nsteps, _W).astype(jnp.int32)
    mesh = plsc.VectorSubcoreMesh(core_axis_name="core", subcore_axis_name="subcore")

    @functools.partial(
        pl.kernel,
        out_type=jax.ShapeDtypeStruct((n, D), emb.dtype),
        mesh=mesh,
        scratch_types=[
            pltpu.VMEM((nsteps, _W), jnp.int32),
            pltpu.VMEM((_NBUF, _W, D), jnp.float32),
            pltpu.SemaphoreType.DMA((_NBUF,)),
            pltpu.SemaphoreType.DMA((_NBUF,)),
        ],
        compiler_params=pltpu.CompilerParams(use_tc_tiling_on_sc=False),
    )
    def gather_kernel(emb_hbm, idx_hbm, out_hbm, idx_v, rows_v, gsem, ssem):
        wid = lax.axis_index("subcore") * info.num_cores + lax.axis_index("core")
        base = wid * n_per_w
        pltpu.sync_copy(idx_hbm.at[wid], idx_v)

        def gather(j, b):
            return pltpu.make_async_copy(
                emb_hbm.at[idx_v.at[j]], rows_v.at[b], gsem.at[b])

        def scatter(j, b):
            return pltpu.make_async_copy(
                rows_v.at[b], out_hbm.at[pl.ds(base + j * _W, _W)], ssem.at[b])

        for b in range(_NBUF):
            gather(b, b).start()

        @pl.loop(0, nsteps - _NBUF, step=_NBUF)
        def _(j0):
            for b in range(_NBUF):
                gather(j0 + b, b).wait()
                scatter(j0 + b, b).start()
            for b in range(_NBUF):
                scatter(j0 + b, b).wait()
                gather(j0 + _NBUF + b, b).start()

        j0 = nsteps - _NBUF
        for b in range(_NBUF):
            gather(j0 + b, b).wait()
            scatter(j0 + b, b).start()
        for b in range(_NBUF):
            scatter(j0 + b, b).wait()

    out = gather_kernel(emb, idx3)
    return out.reshape(B, H, D)
